# Initial kernel scaffold; baseline (speedup 1.0000x reference)
import jax
import jax.numpy as jnp
from jax import lax
from jax.experimental import pallas as pl
from jax.experimental.pallas import tpu as pltpu
from jax.experimental.pallas import tpu_sc as plsc

BATCH = 64
SEQ_LEN = 512
N_EMBD = 512

NC = 2
NS = 16
L = 16
NW = NC * NS
P_PER_W = SEQ_LEN // NW
NBUF = 8
CCHUNKS = N_EMBD // L


def _embed_body(x_hbm, tok_hbm, pos_hbm, out_hbm,
                idx_v, pos_v, gbuf, gsem, osem):
    wid = lax.axis_index("s") * NC + lax.axis_index("c")
    p0 = wid * P_PER_W

    pltpu.sync_copy(x_hbm.at[:, pl.ds(p0, P_PER_W)], idx_v)
    pltpu.sync_copy(pos_hbm.at[pl.ds(p0, P_PER_W), :], pos_v)

    def gather(b, slot):
        pltpu.async_copy(tok_hbm.at[idx_v.at[b]], gbuf.at[slot],
                         gsem.at[slot])

    def out_dma(b, slot):
        return pltpu.make_async_copy(
            gbuf.at[slot], out_hbm.at[b, pl.ds(p0, P_PER_W), :],
            osem.at[slot])

    for k in range(NBUF // 2):
        gather(k, k)

    def group(g, _):
        for k in range(NBUF):
            b = g * NBUF + k
            pltpu.make_async_copy(tok_hbm.at[idx_v.at[b]], gbuf.at[k],
                                  gsem.at[k]).wait()

            def add_chunk(c, _):
                cs = pl.ds(c * L, L)
                for p in range(P_PER_W):
                    gbuf[k, p, cs] = gbuf[k, p, cs] + pos_v[p, cs]
                return ()
            lax.fori_loop(0, CCHUNKS, add_chunk, ())

            out_dma(b, k).start()

            kg = (k + NBUF // 2) % NBUF

            @pl.when(b + NBUF // 2 < BATCH)
            def _():
                @pl.when(b >= NBUF // 2)
                def _():
                    out_dma(b - NBUF // 2, kg).wait()
                gather(b + NBUF // 2, kg)
        return ()

    lax.fori_loop(0, BATCH // NBUF, group, ())

    for k in range(NBUF // 2, NBUF):
        out_dma(BATCH - NBUF + k, k).wait()


@jax.jit
def _embed(x, token_table, position_table):
    mesh = plsc.VectorSubcoreMesh(core_axis_name="c", subcore_axis_name="s")
    return pl.kernel(
        _embed_body,
        out_type=jax.ShapeDtypeStruct((BATCH, SEQ_LEN, N_EMBD), jnp.float32),
        mesh=mesh,
        scratch_types=[
            pltpu.VMEM((BATCH, P_PER_W), jnp.int32),
            pltpu.VMEM((P_PER_W, N_EMBD), jnp.float32),
            pltpu.VMEM((NBUF, P_PER_W, N_EMBD), jnp.float32),
            pltpu.SemaphoreType.DMA((NBUF,)),
            pltpu.SemaphoreType.DMA((NBUF,)),
        ],
    )(x, token_table, position_table)


def kernel(x, token_table, position_table):
    return _embed(x, token_table, position_table)


# speedup vs baseline: 2.9773x; 2.9773x over previous
import jax
import jax.numpy as jnp
from jax import lax
from jax.experimental import pallas as pl
from jax.experimental.pallas import tpu as pltpu
from jax.experimental.pallas import tpu_sc as plsc

BATCH = 64
SEQ_LEN = 512
N_EMBD = 512

NC = 2
NS = 16
L = 16
NW = NC * NS
P_PER_W = SEQ_LEN // NW
NBUF = 8
CCHUNKS = N_EMBD // L


def _embed_body(x_hbm, tok_hbm, pos_hbm, out_hbm,
                idx_v, pos_v, gbuf, gsem, osem):
    wid = lax.axis_index("s") * NC + lax.axis_index("c")
    p0 = wid * P_PER_W

    c0 = (wid // 8) * 128
    coff = (wid % 8) * P_PER_W
    pltpu.sync_copy(x_hbm.at[:, pl.ds(c0, 128)], idx_v)
    pltpu.sync_copy(pos_hbm.at[pl.ds(p0, P_PER_W), :], pos_v)

    def gather(b, slot):
        pltpu.async_copy(tok_hbm.at[idx_v.at[b, pl.ds(coff, P_PER_W)]],
                         gbuf.at[slot], gsem.at[slot])

    def out_dma(b, slot):
        return pltpu.make_async_copy(
            gbuf.at[slot], out_hbm.at[b, pl.ds(p0, P_PER_W), :],
            osem.at[slot])

    for k in range(NBUF // 2):
        gather(k, k)

    def group(g, _):
        for k in range(NBUF):
            b = g * NBUF + k
            pltpu.make_async_copy(
                tok_hbm.at[idx_v.at[b, pl.ds(coff, P_PER_W)]],
                gbuf.at[k], gsem.at[k]).wait()

            def add_chunk(c, _):
                cs = pl.ds(c * L, L)
                for p in range(P_PER_W):
                    gbuf[k, p, cs] = gbuf[k, p, cs] + pos_v[p, cs]
                return ()
            lax.fori_loop(0, CCHUNKS, add_chunk, ())

            out_dma(b, k).start()

            kg = (k + NBUF // 2) % NBUF

            @pl.when(b + NBUF // 2 < BATCH)
            def _():
                @pl.when(b >= NBUF // 2)
                def _():
                    out_dma(b - NBUF // 2, kg).wait()
                gather(b + NBUF // 2, kg)
        return ()

    lax.fori_loop(0, BATCH // NBUF, group, ())

    for k in range(NBUF // 2, NBUF):
        out_dma(BATCH - NBUF + k, k).wait()


@jax.jit
def _embed(x, token_table, position_table):
    mesh = plsc.VectorSubcoreMesh(core_axis_name="c", subcore_axis_name="s")
    return pl.kernel(
        _embed_body,
        out_type=jax.ShapeDtypeStruct((BATCH, SEQ_LEN, N_EMBD), jnp.float32),
        mesh=mesh,
        scratch_types=[
            pltpu.VMEM((BATCH, 128), jnp.int32),
            pltpu.VMEM((P_PER_W, N_EMBD), jnp.float32),
            pltpu.VMEM((NBUF, P_PER_W, N_EMBD), jnp.float32),
            pltpu.SemaphoreType.DMA((NBUF,)),
            pltpu.SemaphoreType.DMA((NBUF,)),
        ],
    )(x, token_table, position_table)


def kernel(x, token_table, position_table):
    return _embed(x, token_table, position_table)
